# baseline (speedup 1.0000x reference)
import jax
import jax.numpy as jnp
from jax import lax
from jax.experimental import pallas as pl
from jax.experimental.pallas import tpu as pltpu
from jax.experimental.pallas import tpu_sc as plsc

_CT_COLS = 4096
_CT_NBLK = 25
_IND_CHUNK = 128

_NW = 32
_SC_SLOTS_PER_W = 2
_PARTS = 4
_PART_ROWS = 16
_NBUF = 3
_IND_WORKERS = 25
_IND_PER_W = 4000

_TC_SLOT_BLK = 8
_TC_SV_STEPS = 8


def _tc_body(s_ref, ct_ref, sv_ref, new_ref, t_ref, ind_ref,
             ct_out, sv_out, chunk_smem, sem):
    i = pl.program_id(0)
    idx = s_ref[0]
    si = s_ref[1]

    @pl.when(i == 0)
    def _():
        base = pl.multiple_of((idx // _IND_CHUNK) * _IND_CHUNK, _IND_CHUNK)
        cp = pltpu.make_async_copy(
            ind_ref.at[pl.ds(base, _IND_CHUNK)], chunk_smem, sem)
        cp.start()
        cp.wait()

    base = pl.multiple_of((idx // _IND_CHUNK) * _IND_CHUNK, _IND_CHUNK)
    cursor = chunk_smem[idx - base]

    b_ct = jnp.minimum(i, _CT_NBLK - 1)
    hit_ct = (idx // _CT_COLS) == b_ct
    cc = idx - b_ct * _CT_COLS
    row_i = jax.lax.broadcasted_iota(jnp.int32, ct_ref.shape, 0)
    col_i = jax.lax.broadcasted_iota(jnp.int32, ct_ref.shape, 1)
    ct_out[...] = jnp.where(hit_ct & (row_i == cursor) & (col_i == cc),
                            t_ref[0], ct_ref[...])

    b_sv = _TC_SLOT_BLK + jnp.clip(i - _CT_NBLK, 0, _TC_SV_STEPS - 1)
    slot_i = (jax.lax.broadcasted_iota(jnp.int32, sv_ref.shape, 0)
              + b_sv * _TC_SLOT_BLK)
    sv_out[...] = jnp.where(slot_i == si, new_ref[...], sv_ref[...])


def _sc_body(idx16_ref, si16_ref, ind_ref, saved_ref, new_ref,
             ind_out, saved_out, bufs, indbuf, sbuf, in_sems, out_sems):
    wid = lax.axis_index("s") * 2 + lax.axis_index("c")

    pltpu.sync_copy(idx16_ref, sbuf)
    idx = jnp.max(sbuf[...])
    pltpu.sync_copy(si16_ref, sbuf)
    si = jnp.max(sbuf[...])

    n_chunks = _SC_SLOTS_PER_W * _PARTS

    def chunk_coords(j):
        slot = wid * _SC_SLOTS_PER_W + (j // _PARTS)
        part = (j % _PARTS) * _PART_ROWS
        return slot, part

    def start_in(j):
        slot, part = chunk_coords(j)
        buf = bufs.at[j % _NBUF]
        sem = in_sems.at[j % _NBUF]

        @pl.when(slot == si)
        def _():
            pltpu.make_async_copy(
                new_ref.at[slot, pl.ds(part, _PART_ROWS), :], buf, sem
            ).start()

        @pl.when(slot != si)
        def _():
            pltpu.make_async_copy(
                saved_ref.at[slot, pl.ds(part, _PART_ROWS), :], buf, sem
            ).start()

    def wait_in(j):
        slot, part = chunk_coords(j)
        pltpu.make_async_copy(
            saved_ref.at[slot, pl.ds(part, _PART_ROWS), :],
            bufs.at[j % _NBUF], in_sems.at[j % _NBUF]).wait()

    def out_cp(j):
        slot, part = chunk_coords(j)
        return pltpu.make_async_copy(
            bufs.at[j % _NBUF],
            saved_out.at[slot, pl.ds(part, _PART_ROWS), :],
            out_sems.at[j % _NBUF])

    for j in range(_NBUF):
        start_in(j)
    for k in range(n_chunks):
        wait_in(k)
        out_cp(k).start()
        if k >= 1:
            out_cp(k - 1).wait()
            j = k - 1 + _NBUF
            if j < n_chunks:
                start_in(j)
    out_cp(n_chunks - 1).wait()

    @pl.when(wid < _IND_WORKERS)
    def _():
        base = wid * _IND_PER_W
        pltpu.sync_copy(ind_ref.at[pl.ds(base, _IND_PER_W)], indbuf)

        @pl.when(wid == idx // _IND_PER_W)
        def _():
            g = pl.multiple_of((idx // 16) * 16 - base, 8)
            v = indbuf[pl.ds(g, 16)]
            lane = idx - (idx // 16) * 16
            indbuf[pl.ds(g, 16)] = jnp.where(
                lax.iota(jnp.int32, 16) == lane, v + 1, v)

        pltpu.sync_copy(indbuf, ind_out.at[pl.ds(base, _IND_PER_W)])


def kernel(clicktimes, indices, idx, t, saved, new, save_index):
    idx32 = jnp.asarray(idx, jnp.int32)
    si32 = jnp.asarray(save_index, jnp.int32)
    t_arr = jnp.asarray(t, jnp.float32).reshape(1)
    idx16 = jnp.full((16,), idx32, jnp.int32)
    si16 = jnp.full((16,), si32, jnp.int32)

    ct_t = clicktimes.T
    saved_t = saved.transpose(0, 2, 1)
    new_t = new.transpose(0, 2, 1)

    mesh = plsc.VectorSubcoreMesh(core_axis_name="c", subcore_axis_name="s")
    sc_fn = pl.kernel(
        _sc_body,
        out_type=[
            jax.ShapeDtypeStruct(indices.shape, indices.dtype),
            jax.ShapeDtypeStruct(saved_t.shape, saved_t.dtype),
        ],
        mesh=mesh,
        scratch_types=[
            pltpu.VMEM((_NBUF, _PART_ROWS, saved_t.shape[2]), saved_t.dtype),
            pltpu.VMEM((_IND_PER_W,), indices.dtype),
            pltpu.VMEM((16,), jnp.int32),
            pltpu.SemaphoreType.DMA((_NBUF,)),
            pltpu.SemaphoreType.DMA((_NBUF,)),
        ],
        compiler_params=pltpu.CompilerParams(needs_layout_passes=False),
    )
    ind_out, saved_mid = sc_fn(idx16, si16, indices, saved_t, new_t)

    n_clicks = ct_t.shape[0]
    sv_blk = (_TC_SLOT_BLK,) + saved_t.shape[1:]
    grid_spec = pltpu.PrefetchScalarGridSpec(
        num_scalar_prefetch=1,
        grid=(_CT_NBLK + _TC_SV_STEPS,),
        in_specs=[
            pl.BlockSpec((n_clicks, _CT_COLS),
                         lambda i, s: (0, jnp.minimum(i, _CT_NBLK - 1))),
            pl.BlockSpec(sv_blk,
                         lambda i, s: (_TC_SLOT_BLK
                                       + jnp.clip(i - _CT_NBLK, 0,
                                                  _TC_SV_STEPS - 1), 0, 0)),
            pl.BlockSpec((1,) + saved_t.shape[1:],
                         lambda i, s: (s[1], 0, 0)),
            pl.BlockSpec(memory_space=pltpu.SMEM),
            pl.BlockSpec(memory_space=pltpu.HBM),
        ],
        out_specs=[
            pl.BlockSpec((n_clicks, _CT_COLS),
                         lambda i, s: (0, jnp.minimum(i, _CT_NBLK - 1))),
            pl.BlockSpec(sv_blk,
                         lambda i, s: (_TC_SLOT_BLK
                                       + jnp.clip(i - _CT_NBLK, 0,
                                                  _TC_SV_STEPS - 1), 0, 0)),
        ],
        scratch_shapes=[
            pltpu.SMEM((_IND_CHUNK,), indices.dtype),
            pltpu.SemaphoreType.DMA,
        ],
    )
    ct_out_t, saved_out_t = pl.pallas_call(
        _tc_body,
        grid_spec=grid_spec,
        out_shape=[
            jax.ShapeDtypeStruct(ct_t.shape, ct_t.dtype),
            jax.ShapeDtypeStruct(saved_t.shape, saved_t.dtype),
        ],
        input_output_aliases={1: 0, 2: 1},
        compiler_params=pltpu.CompilerParams(
            dimension_semantics=("arbitrary",)),
    )(jnp.stack([idx32, si32]), ct_t, saved_mid, new_t, t_arr, indices)

    return (ct_out_t.T, ind_out, saved_out_t.transpose(0, 2, 1),
            save_index + 1)


# speedup vs baseline: 1.4808x; 1.4808x over previous
import jax
import jax.numpy as jnp
from jax import lax
from jax.experimental import pallas as pl
from jax.experimental.pallas import tpu as pltpu
from jax.experimental.pallas import tpu_sc as plsc

_CT_COLS = 128
_IND_CHUNK = 128

_NW = 32
_SLOTS_PER_W = 4
_PARTS = 4
_PART_ROWS = 16
_NBUF = 3
_IND_WORKERS = 25
_IND_PER_W = 4000


def _tc_body(s_ref, ct_ref, ind_ref, t_ref, ct_out, chunk_smem, sem):
    idx = s_ref[0]

    base = pl.multiple_of((idx // _IND_CHUNK) * _IND_CHUNK, _IND_CHUNK)
    cur_cp = pltpu.make_async_copy(
        ind_ref.at[pl.ds(base, _IND_CHUNK)], chunk_smem, sem)
    cur_cp.start()
    cur_cp.wait()
    cursor = chunk_smem[idx - base]

    cc = idx - (idx // _CT_COLS) * _CT_COLS
    row_i = jax.lax.broadcasted_iota(jnp.int32, ct_ref.shape, 0)
    col_i = jax.lax.broadcasted_iota(jnp.int32, ct_ref.shape, 1)
    ct_out[...] = jnp.where((row_i == cursor) & (col_i == cc),
                            t_ref[0], ct_ref[...])


def _sc_body(idx16_ref, si16_ref, ind_ref, saved_ref, new_ref,
             ind_out, saved_out, bufs, indbuf, sbuf, in_sems, out_sems):
    wid = lax.axis_index("s") * 2 + lax.axis_index("c")

    pltpu.sync_copy(idx16_ref, sbuf)
    idx = jnp.max(sbuf[...])
    pltpu.sync_copy(si16_ref, sbuf)
    si = jnp.max(sbuf[...])

    n_chunks = _SLOTS_PER_W * _PARTS

    def chunk_coords(j):
        slot = wid * _SLOTS_PER_W + (j // _PARTS)
        part = (j % _PARTS) * _PART_ROWS
        return slot, part

    def start_in(j):
        slot, part = chunk_coords(j)
        buf = bufs.at[j % _NBUF]
        sem = in_sems.at[j % _NBUF]

        @pl.when(slot == si)
        def _():
            pltpu.make_async_copy(
                new_ref.at[slot, pl.ds(part, _PART_ROWS), :], buf, sem
            ).start()

        @pl.when(slot != si)
        def _():
            pltpu.make_async_copy(
                saved_ref.at[slot, pl.ds(part, _PART_ROWS), :], buf, sem
            ).start()

    def wait_in(j):
        slot, part = chunk_coords(j)
        pltpu.make_async_copy(
            saved_ref.at[slot, pl.ds(part, _PART_ROWS), :],
            bufs.at[j % _NBUF], in_sems.at[j % _NBUF]).wait()

    def out_cp(j):
        slot, part = chunk_coords(j)
        return pltpu.make_async_copy(
            bufs.at[j % _NBUF],
            saved_out.at[slot, pl.ds(part, _PART_ROWS), :],
            out_sems.at[j % _NBUF])

    for j in range(_NBUF):
        start_in(j)
    for k in range(n_chunks):
        wait_in(k)
        out_cp(k).start()
        if k >= 1:
            out_cp(k - 1).wait()
            j = k - 1 + _NBUF
            if j < n_chunks:
                start_in(j)
    out_cp(n_chunks - 1).wait()

    @pl.when(wid < _IND_WORKERS)
    def _():
        base = wid * _IND_PER_W
        pltpu.sync_copy(ind_ref.at[pl.ds(base, _IND_PER_W)], indbuf)

        @pl.when(wid == idx // _IND_PER_W)
        def _():
            g = pl.multiple_of((idx // 16) * 16 - base, 8)
            v = indbuf[pl.ds(g, 16)]
            lane = idx - (idx // 16) * 16
            indbuf[pl.ds(g, 16)] = jnp.where(
                lax.iota(jnp.int32, 16) == lane, v + 1, v)

        pltpu.sync_copy(indbuf, ind_out.at[pl.ds(base, _IND_PER_W)])


def kernel(clicktimes, indices, idx, t, saved, new, save_index):
    idx32 = jnp.asarray(idx, jnp.int32)
    si32 = jnp.asarray(save_index, jnp.int32)
    t_arr = jnp.asarray(t, jnp.float32).reshape(1)
    idx16 = jnp.full((16,), idx32, jnp.int32)
    si16 = jnp.full((16,), si32, jnp.int32)

    ct_t = clicktimes.T
    saved_t = saved.transpose(0, 2, 1)
    new_t = new.transpose(0, 2, 1)

    mesh = plsc.VectorSubcoreMesh(core_axis_name="c", subcore_axis_name="s")
    sc_fn = pl.kernel(
        _sc_body,
        out_type=[
            jax.ShapeDtypeStruct(indices.shape, indices.dtype),
            jax.ShapeDtypeStruct(saved_t.shape, saved_t.dtype),
        ],
        mesh=mesh,
        scratch_types=[
            pltpu.VMEM((_NBUF, _PART_ROWS, saved_t.shape[2]), saved_t.dtype),
            pltpu.VMEM((_IND_PER_W,), indices.dtype),
            pltpu.VMEM((16,), jnp.int32),
            pltpu.SemaphoreType.DMA((_NBUF,)),
            pltpu.SemaphoreType.DMA((_NBUF,)),
        ],
        compiler_params=pltpu.CompilerParams(needs_layout_passes=False,
                                             skip_device_barrier=True),
    )
    ind_out, saved_out_t = sc_fn(idx16, si16, indices, saved_t, new_t)

    n_clicks = ct_t.shape[0]
    grid_spec = pltpu.PrefetchScalarGridSpec(
        num_scalar_prefetch=1,
        grid=(1,),
        in_specs=[
            pl.BlockSpec((n_clicks, _CT_COLS),
                         lambda i, s: (0, s[0] // _CT_COLS)),
            pl.BlockSpec(memory_space=pltpu.HBM),
            pl.BlockSpec(memory_space=pltpu.SMEM),
        ],
        out_specs=[
            pl.BlockSpec((n_clicks, _CT_COLS),
                         lambda i, s: (0, s[0] // _CT_COLS)),
        ],
        scratch_shapes=[
            pltpu.SMEM((_IND_CHUNK,), indices.dtype),
            pltpu.SemaphoreType.DMA,
        ],
    )
    (ct_out_t,) = pl.pallas_call(
        _tc_body,
        grid_spec=grid_spec,
        out_shape=[jax.ShapeDtypeStruct(ct_t.shape, ct_t.dtype)],
        input_output_aliases={1: 0},
        compiler_params=pltpu.CompilerParams(skip_device_barrier=True),
    )(jnp.stack([idx32]), ct_t, indices, t_arr)

    return (ct_out_t.T, ind_out, saved_out_t.transpose(0, 2, 1),
            save_index + 1)


# speedup vs baseline: 1.7728x; 1.1972x over previous
import jax
import jax.numpy as jnp
from jax.experimental import pallas as pl
from jax.experimental.pallas import tpu as pltpu

_CT_COLS = 128
_IND_CHUNK = 128


def _body(s_ref, ct_ref, ind_ref, t_ref, saved_ref, new_ref,
          ct_out, ind_out, saved_out, chunk_smem, sem):
    del saved_ref
    idx = s_ref[0]

    base = pl.multiple_of((idx // _IND_CHUNK) * _IND_CHUNK, _IND_CHUNK)
    cur_cp = pltpu.make_async_copy(
        ind_ref.at[pl.ds(base, _IND_CHUNK)], chunk_smem, sem)
    cur_cp.start()
    cur_cp.wait()
    off = idx - base
    cursor = chunk_smem[off]

    chunk_smem[off] = cursor + 1
    ind_fix = pltpu.make_async_copy(
        chunk_smem, ind_out.at[pl.ds(base, _IND_CHUNK)], sem)
    ind_fix.start()

    cc = idx - (idx // _CT_COLS) * _CT_COLS
    row_i = jax.lax.broadcasted_iota(jnp.int32, ct_ref.shape, 0)
    col_i = jax.lax.broadcasted_iota(jnp.int32, ct_ref.shape, 1)
    ct_out[...] = jnp.where((row_i == cursor) & (col_i == cc),
                            t_ref[0], ct_ref[...])

    saved_out[...] = new_ref[...]

    ind_fix.wait()


def kernel(clicktimes, indices, idx, t, saved, new, save_index):
    idx32 = jnp.asarray(idx, jnp.int32)
    si32 = jnp.asarray(save_index, jnp.int32)
    s = jnp.stack([idx32, si32])
    t_arr = jnp.asarray(t, jnp.float32).reshape(1)

    ct_t = clicktimes.T
    saved_t = saved.transpose(0, 2, 1)
    new_t = new.transpose(0, 2, 1)

    n_clicks = ct_t.shape[0]
    slot_blk = (1,) + saved_t.shape[1:]
    grid_spec = pltpu.PrefetchScalarGridSpec(
        num_scalar_prefetch=1,
        grid=(1,),
        in_specs=[
            pl.BlockSpec((n_clicks, _CT_COLS),
                         lambda i, s: (0, s[0] // _CT_COLS)),
            pl.BlockSpec(memory_space=pltpu.HBM),
            pl.BlockSpec(memory_space=pltpu.SMEM),
            pl.BlockSpec(slot_blk, lambda i, s: (s[1], 0, 0)),
            pl.BlockSpec(slot_blk, lambda i, s: (s[1], 0, 0)),
        ],
        out_specs=[
            pl.BlockSpec((n_clicks, _CT_COLS),
                         lambda i, s: (0, s[0] // _CT_COLS)),
            pl.BlockSpec(memory_space=pltpu.HBM),
            pl.BlockSpec(slot_blk, lambda i, s: (s[1], 0, 0)),
        ],
        scratch_shapes=[
            pltpu.SMEM((_IND_CHUNK,), indices.dtype),
            pltpu.SemaphoreType.DMA,
        ],
    )
    ct_out_t, ind_out, saved_out_t = pl.pallas_call(
        _body,
        grid_spec=grid_spec,
        out_shape=[
            jax.ShapeDtypeStruct(ct_t.shape, ct_t.dtype),
            jax.ShapeDtypeStruct(indices.shape, indices.dtype),
            jax.ShapeDtypeStruct(saved_t.shape, saved_t.dtype),
        ],
        input_output_aliases={1: 0, 2: 1, 4: 2},
    )(s, ct_t, indices, t_arr, saved_t, new_t)

    return (ct_out_t.T, ind_out, saved_out_t.transpose(0, 2, 1),
            save_index + 1)
